# baseline (speedup 1.0000x reference)
import functools

import jax
import jax.numpy as jnp
from jax import lax
from jax.experimental import pallas as pl
from jax.experimental.pallas import tpu as pltpu
from jax.experimental.pallas import tpu_sc as plsc

B = 1024
N_NUM = 13
N_CAT = 26
L = 5
VH = 30522
VV = 100000
D = 768
H = 128

NC = 2
NS = 16
NW = NC * NS

CS = 64
SV = B * N_CAT
SH = 28672
NV = SV // NW
NH = SH // NW
VPV = 100800
VPH = 31200
BLK = 800


def _table_proj_body(tab_ref, w_ref, b_ref, wt_ref, o_ref, *, v_rows):
    pid = pl.program_id(0)
    x = tab_ref[...]
    mu = jnp.mean(x, axis=1, keepdims=True)
    var = jnp.mean((x - mu) ** 2, axis=1, keepdims=True)
    y = (x - mu) * lax.rsqrt(var + 1e-5) * w_ref[...] + b_ref[...]
    z = jnp.dot(y, wt_ref[...], preferred_element_type=jnp.float32)
    rows = pid * BLK + lax.broadcasted_iota(jnp.int32, (BLK, 1), 0)
    o_ref[...] = jnp.where(rows < v_rows, z, 0.0)


def _table_proj(tab, w, b, wt, v_rows, vp_rows):
    nb = vp_rows // BLK
    in_cap = (v_rows + BLK - 1) // BLK - 1
    f = pl.pallas_call(
        functools.partial(_table_proj_body, v_rows=v_rows),
        grid=(nb,),
        in_specs=[
            pl.BlockSpec((BLK, D), lambda i: (jnp.minimum(i, in_cap), 0)),
            pl.BlockSpec((1, D), lambda i: (0, 0)),
            pl.BlockSpec((1, D), lambda i: (0, 0)),
            pl.BlockSpec((D, H), lambda i: (0, 0)),
        ],
        out_specs=pl.BlockSpec((BLK, H), lambda i: (i, 0)),
        out_shape=jax.ShapeDtypeStruct((vp_rows, H), jnp.float32),
    )
    return f(tab, w.reshape(1, D), b.reshape(1, D), wt)


def _sc_pool_body(gv_hbm, gh_hbm, idv_hbm, idh_hbm, ov_hbm, oh_hbm,
                  ids_v, rows_v, acc_v, sem):
    wid = lax.axis_index("s") * NC + lax.axis_index("c")

    def run_side(g_hbm, id_hbm, o_hbm, seg_total, n_per_w):
        base = wid * n_per_w
        for l in range(L):
            pltpu.sync_copy(id_hbm.at[pl.ds(l * seg_total + base, n_per_w)],
                            ids_v.at[l, pl.ds(0, n_per_w)])

        def chunk(ci, _):
            off = ci * CS
            descs = [
                pltpu.make_async_copy(
                    g_hbm.at[ids_v.at[l, pl.ds(off, CS)]], rows_v.at[l], sem)
                for l in range(L)
            ]
            for dsc in descs:
                dsc.start()
            for dsc in descs:
                dsc.wait()

            def seg(si, _):
                for dc in range(H // 16):
                    sl = pl.ds(dc * 16, 16)
                    acc = rows_v[0, si, sl] + rows_v[1, si, sl]
                    acc = acc + rows_v[2, si, sl]
                    acc = acc + rows_v[3, si, sl]
                    acc = acc + rows_v[4, si, sl]
                    acc_v[si, sl] = acc
                return 0

            lax.fori_loop(0, CS, seg, 0)
            pltpu.sync_copy(acc_v, o_hbm.at[pl.ds(base + off, CS)])
            return 0

        lax.fori_loop(0, n_per_w // CS, chunk, 0)

    run_side(gv_hbm, idv_hbm, ov_hbm, SV, NV)
    run_side(gh_hbm, idh_hbm, oh_hbm, SH, NH)


@functools.cache
def _sc_pool():
    return pl.kernel(
        _sc_pool_body,
        mesh=plsc.VectorSubcoreMesh(core_axis_name="c", subcore_axis_name="s"),
        out_type=[
            jax.ShapeDtypeStruct((SV, H), jnp.float32),
            jax.ShapeDtypeStruct((SH, H), jnp.float32),
        ],
        scratch_types=[
            pltpu.VMEM((L, NH), jnp.int32),
            pltpu.VMEM((L, CS, H), jnp.float32),
            pltpu.VMEM((CS, H), jnp.float32),
            pltpu.SemaphoreType.DMA,
        ],
        compiler_params=pltpu.CompilerParams(use_tc_tiling_on_sc=False),
    )


def _assemble_body(xnum_ref, vsum_ref, hsum_ref, xm_ref, nm_ref, cm_ref,
                   ncp_ref, colp_ref, nbias_ref, wt_ref, emb_ref, bert_ref):
    eps = 1e-12
    nden = jnp.sum(nm_ref[...].astype(jnp.float32), axis=1, keepdims=True) + eps
    ncp_avg = ncp_ref[...] / nden
    cden = jnp.sum(cm_ref[...].astype(jnp.float32), axis=1, keepdims=True) + eps
    colp_avg = colp_ref[...] / cden
    bias_p = jnp.dot(nbias_ref[...], wt_ref[...],
                     preferred_element_type=jnp.float32)
    rden = 1.0 / (jnp.sum(xm_ref[...].astype(jnp.float32), axis=2) + eps)
    val_avg = vsum_ref[...] * rden[:, :, None]
    hdr_avg = hsum_ref[...] * rden[:, :, None]
    num_part = xnum_ref[...][:, :, None] * ncp_avg[None] + bias_p[None]
    cat_part = (colp_avg[None] + val_avg) * 0.5
    emb_ref[...] = jnp.concatenate([num_part, cat_part], axis=1)
    bert_ref[...] = hdr_avg


def _assemble(xnum, vsum, hsum, xm, nm, cm, ncp, colp, nbias, wt):
    BB = 128
    nb = B // BB
    f = pl.pallas_call(
        _assemble_body,
        grid=(nb,),
        in_specs=[
            pl.BlockSpec((BB, N_NUM), lambda i: (i, 0)),
            pl.BlockSpec((BB, N_CAT, H), lambda i: (i, 0, 0)),
            pl.BlockSpec((BB, N_CAT, H), lambda i: (i, 0, 0)),
            pl.BlockSpec((BB, N_CAT, L), lambda i: (i, 0, 0)),
            pl.BlockSpec((N_NUM, L), lambda i: (0, 0)),
            pl.BlockSpec((N_CAT, L), lambda i: (0, 0)),
            pl.BlockSpec((N_NUM, H), lambda i: (0, 0)),
            pl.BlockSpec((N_CAT, H), lambda i: (0, 0)),
            pl.BlockSpec((1, D), lambda i: (0, 0)),
            pl.BlockSpec((D, H), lambda i: (0, 0)),
        ],
        out_specs=[
            pl.BlockSpec((BB, N_NUM + N_CAT, H), lambda i: (i, 0, 0)),
            pl.BlockSpec((BB, N_CAT, H), lambda i: (i, 0, 0)),
        ],
        out_shape=[
            jax.ShapeDtypeStruct((B, N_NUM + N_CAT, H), jnp.float32),
            jax.ShapeDtypeStruct((B, N_CAT, H), jnp.float32),
        ],
    )
    return f(xnum, vsum, hsum, xm, nm, cm, ncp, colp, nbias, wt)


def kernel(x_num, num_col_input_ids, num_att_mask, x_cat_input_ids,
           x_cat_att_mask, col_cat_input_ids, col_cat_att_mask, header_table,
           value_table, norm_header_w, norm_header_b, norm_value_w,
           norm_value_b, num_bias, align_W):
    wt = align_W.T
    gv = _table_proj(value_table, norm_value_w, norm_value_b, wt, VV, VPV)
    gh = _table_proj(header_table, norm_header_w, norm_header_b, wt, VH, VPH)

    xm = x_cat_att_mask != 0
    val_ids = jnp.where(xm, x_cat_input_ids, VV).reshape(SV, L)
    hdr_ids = jnp.where(xm, x_cat_input_ids, VH).reshape(SV, L)
    nids = jnp.where(num_att_mask != 0, num_col_input_ids, VH)
    cids = jnp.where(col_cat_att_mask != 0, col_cat_input_ids, VH)
    pad = jnp.full((SH - SV - N_NUM - N_CAT, L), VH, jnp.int32)
    hdr_all = jnp.concatenate([hdr_ids, nids, cids, pad], axis=0)
    idv = val_ids.T.reshape(-1)
    idh = hdr_all.T.reshape(-1)

    vsum, hsum_all = _sc_pool()(gv, gh, idv, idh)
    vsum = vsum.reshape(B, N_CAT, H)
    hsum = hsum_all[:SV].reshape(B, N_CAT, H)
    ncp_sum = hsum_all[SV:SV + N_NUM]
    colp_sum = hsum_all[SV + N_NUM:SV + N_NUM + N_CAT]

    emb, bert = _assemble(x_num, vsum, hsum, x_cat_att_mask, num_att_mask,
                          col_cat_att_mask, ncp_sum, colp_sum,
                          num_bias.reshape(1, D), wt)
    return emb, bert


# speedup vs baseline: 11.4180x; 11.4180x over previous
import functools

import jax
import jax.numpy as jnp
from jax import lax
from jax.experimental import pallas as pl
from jax.experimental.pallas import tpu as pltpu
from jax.experimental.pallas import tpu_sc as plsc

B = 1024
N_NUM = 13
N_CAT = 26
L = 5
VH = 30522
VV = 100000
D = 768
H = 128

NC = 2
NS = 16
NW = NC * NS

CS = 64
SV = B * N_CAT
SH = 28672
NV = SV // NW
NH = SH // NW
VPV = 100800
VPH = 31200
BLK = 800


def _table_proj_body(tab_ref, w_ref, b_ref, wt_ref, o_ref, *, v_rows):
    pid = pl.program_id(0)
    x = tab_ref[...]
    mu = jnp.mean(x, axis=1, keepdims=True)
    var = jnp.mean((x - mu) ** 2, axis=1, keepdims=True)
    y = (x - mu) * lax.rsqrt(var + 1e-5) * w_ref[...] + b_ref[...]
    z = jnp.dot(y, wt_ref[...], preferred_element_type=jnp.float32)
    rows = pid * BLK + lax.broadcasted_iota(jnp.int32, (BLK, 1), 0)
    o_ref[...] = jnp.where(rows < v_rows, z, 0.0)


def _table_proj(tab, w, b, wt, v_rows, vp_rows):
    nb = vp_rows // BLK
    in_cap = (v_rows + BLK - 1) // BLK - 1
    f = pl.pallas_call(
        functools.partial(_table_proj_body, v_rows=v_rows),
        grid=(nb,),
        in_specs=[
            pl.BlockSpec((BLK, D), lambda i: (jnp.minimum(i, in_cap), 0)),
            pl.BlockSpec((1, D), lambda i: (0, 0)),
            pl.BlockSpec((1, D), lambda i: (0, 0)),
            pl.BlockSpec((D, H), lambda i: (0, 0)),
        ],
        out_specs=pl.BlockSpec((BLK, H), lambda i: (i, 0)),
        out_shape=jax.ShapeDtypeStruct((vp_rows, H), jnp.float32),
    )
    return f(tab, w.reshape(1, D), b.reshape(1, D), wt)


def _sc_pool_body(gv_hbm, gh_hbm, idv_hbm, idh_hbm, ov_hbm, oh_hbm,
                  ids_v, rows_v, acc_v, sem):
    wid = lax.axis_index("s") * NC + lax.axis_index("c")

    def run_side(g_hbm, id_hbm, o_hbm, seg_total, n_per_w):
        base = wid * n_per_w
        for l in range(L):
            pltpu.sync_copy(id_hbm.at[pl.ds(l * seg_total + base, n_per_w)],
                            ids_v.at[l, pl.ds(0, n_per_w)])

        def chunk(ci, _):
            off = ci * CS
            descs = [
                pltpu.make_async_copy(
                    g_hbm.at[ids_v.at[l, pl.ds(off, CS)]], rows_v.at[l], sem)
                for l in range(L)
            ]
            for dsc in descs:
                dsc.start()
            for dsc in descs:
                dsc.wait()

            def seg(si, _):
                for dc in range(H // 16):
                    sl = pl.ds(dc * 16, 16)
                    acc = rows_v[0, si, sl] + rows_v[1, si, sl]
                    acc = acc + rows_v[2, si, sl]
                    acc = acc + rows_v[3, si, sl]
                    acc = acc + rows_v[4, si, sl]
                    acc_v[si, sl] = acc
                return 0

            lax.fori_loop(0, CS, seg, 0)
            pltpu.sync_copy(acc_v, o_hbm.at[pl.ds(base + off, CS)])
            return 0

        lax.fori_loop(0, n_per_w // CS, chunk, 0)

    run_side(gv_hbm, idv_hbm, ov_hbm, SV, NV)
    run_side(gh_hbm, idh_hbm, oh_hbm, SH, NH)


@functools.cache
def _sc_pool():
    return pl.kernel(
        _sc_pool_body,
        mesh=plsc.VectorSubcoreMesh(core_axis_name="c", subcore_axis_name="s"),
        out_type=[
            jax.ShapeDtypeStruct((SV, H), jnp.float32),
            jax.ShapeDtypeStruct((SH, H), jnp.float32),
        ],
        scratch_types=[
            pltpu.VMEM((L, NH), jnp.int32),
            pltpu.VMEM((L, CS, H), jnp.float32),
            pltpu.VMEM((CS, H), jnp.float32),
            pltpu.SemaphoreType.DMA,
        ],
        compiler_params=pltpu.CompilerParams(use_tc_tiling_on_sc=False),
    )


def _assemble_body(xnum_ref, vsum_ref, hsum_ref, xm_ref, nm_ref, cm_ref,
                   ncp_ref, colp_ref, nbias_ref, wt_ref, emb_ref, bert_ref):
    eps = 1e-12
    nden = jnp.sum(nm_ref[...].astype(jnp.float32), axis=1, keepdims=True) + eps
    ncp_avg = ncp_ref[...] / nden
    cden = jnp.sum(cm_ref[...].astype(jnp.float32), axis=1, keepdims=True) + eps
    colp_avg = colp_ref[...] / cden
    bias_p = jnp.dot(nbias_ref[...], wt_ref[...],
                     preferred_element_type=jnp.float32)
    rden = 1.0 / (jnp.sum(xm_ref[...].astype(jnp.float32), axis=2) + eps)
    val_avg = vsum_ref[...] * rden[:, :, None]
    hdr_avg = hsum_ref[...] * rden[:, :, None]
    num_part = xnum_ref[...][:, :, None] * ncp_avg[None] + bias_p[None]
    cat_part = (colp_avg[None] + val_avg) * 0.5
    emb_ref[...] = jnp.concatenate([num_part, cat_part], axis=1)
    bert_ref[...] = hdr_avg


def _assemble(xnum, vsum, hsum, xm, nm, cm, ncp, colp, nbias, wt):
    BB = 128
    nb = B // BB
    f = pl.pallas_call(
        _assemble_body,
        grid=(nb,),
        in_specs=[
            pl.BlockSpec((BB, N_NUM), lambda i: (i, 0)),
            pl.BlockSpec((BB, N_CAT, H), lambda i: (i, 0, 0)),
            pl.BlockSpec((BB, N_CAT, H), lambda i: (i, 0, 0)),
            pl.BlockSpec((BB, N_CAT, L), lambda i: (i, 0, 0)),
            pl.BlockSpec((N_NUM, L), lambda i: (0, 0)),
            pl.BlockSpec((N_CAT, L), lambda i: (0, 0)),
            pl.BlockSpec((N_NUM, H), lambda i: (0, 0)),
            pl.BlockSpec((N_CAT, H), lambda i: (0, 0)),
            pl.BlockSpec((1, D), lambda i: (0, 0)),
            pl.BlockSpec((D, H), lambda i: (0, 0)),
        ],
        out_specs=[
            pl.BlockSpec((BB, N_NUM + N_CAT, H), lambda i: (i, 0, 0)),
            pl.BlockSpec((BB, N_CAT, H), lambda i: (i, 0, 0)),
        ],
        out_shape=[
            jax.ShapeDtypeStruct((B, N_NUM + N_CAT, H), jnp.float32),
            jax.ShapeDtypeStruct((B, N_CAT, H), jnp.float32),
        ],
    )
    return f(xnum, vsum, hsum, xm, nm, cm, ncp, colp, nbias, wt)


def kernel(x_num, num_col_input_ids, num_att_mask, x_cat_input_ids,
           x_cat_att_mask, col_cat_input_ids, col_cat_att_mask, header_table,
           value_table, norm_header_w, norm_header_b, norm_value_w,
           norm_value_b, num_bias, align_W):
    wt = align_W.T
    gv = _table_proj(value_table, norm_value_w, norm_value_b, wt, VV, VPV)
    gh = _table_proj(header_table, norm_header_w, norm_header_b, wt, VH, VPH)

    xm = x_cat_att_mask != 0
    spread = (jnp.arange(SV * L, dtype=jnp.int32) & 511).reshape(B, N_CAT, L)
    val_ids = jnp.where(xm, x_cat_input_ids, VV + spread).reshape(SV, L)
    hdr_ids = jnp.where(xm, x_cat_input_ids, VH + spread).reshape(SV, L)
    spread_n = (jnp.arange(N_NUM * L, dtype=jnp.int32) & 511).reshape(N_NUM, L)
    spread_c = (jnp.arange(N_CAT * L, dtype=jnp.int32) & 511).reshape(N_CAT, L)
    nids = jnp.where(num_att_mask != 0, num_col_input_ids, VH + spread_n)
    cids = jnp.where(col_cat_att_mask != 0, col_cat_input_ids, VH + spread_c)
    npad = SH - SV - N_NUM - N_CAT
    pad = VH + (jnp.arange(npad * L, dtype=jnp.int32) & 511).reshape(npad, L)
    hdr_all = jnp.concatenate([hdr_ids, nids, cids, pad], axis=0)
    idv = val_ids.T.reshape(-1)
    idh = hdr_all.T.reshape(-1)

    vsum, hsum_all = _sc_pool()(gv, gh, idv, idh)
    vsum = vsum.reshape(B, N_CAT, H)
    hsum = hsum_all[:SV].reshape(B, N_CAT, H)
    ncp_sum = hsum_all[SV:SV + N_NUM]
    colp_sum = hsum_all[SV + N_NUM:SV + N_NUM + N_CAT]

    emb, bert = _assemble(x_num, vsum, hsum, x_cat_att_mask, num_att_mask,
                          col_cat_att_mask, ncp_sum, colp_sum,
                          num_bias.reshape(1, D), wt)
    return emb, bert


# speedup vs baseline: 12.8781x; 1.1279x over previous
import functools

import jax
import jax.numpy as jnp
from jax import lax
from jax.experimental import pallas as pl
from jax.experimental.pallas import tpu as pltpu
from jax.experimental.pallas import tpu_sc as plsc

B = 1024
N_NUM = 13
N_CAT = 26
L = 5
VH = 30522
VV = 100000
D = 768
H = 128

NC = 2
NS = 16
NW = NC * NS

CS = 64
SV = B * N_CAT
SH = 28672
NV = SV // NW
NH = SH // NW
VPV = 100800
VPH = 31200
BLK = 800


def _table_proj_body(tab_ref, w_ref, b_ref, wt_ref, o_ref, *, v_rows):
    pid = pl.program_id(0)
    x = tab_ref[...]
    mu = jnp.mean(x, axis=1, keepdims=True)
    var = jnp.mean((x - mu) ** 2, axis=1, keepdims=True)
    y = (x - mu) * lax.rsqrt(var + 1e-5) * w_ref[...] + b_ref[...]
    z = jnp.dot(y, wt_ref[...], preferred_element_type=jnp.float32)
    rows = pid * BLK + lax.broadcasted_iota(jnp.int32, (BLK, 1), 0)
    o_ref[...] = jnp.where(rows < v_rows, z, 0.0)


def _table_proj(tab, w, b, wt, v_rows, vp_rows):
    nb = vp_rows // BLK
    in_cap = (v_rows + BLK - 1) // BLK - 1
    f = pl.pallas_call(
        functools.partial(_table_proj_body, v_rows=v_rows),
        grid=(nb,),
        in_specs=[
            pl.BlockSpec((BLK, D), lambda i: (jnp.minimum(i, in_cap), 0)),
            pl.BlockSpec((1, D), lambda i: (0, 0)),
            pl.BlockSpec((1, D), lambda i: (0, 0)),
            pl.BlockSpec((D, H), lambda i: (0, 0)),
        ],
        out_specs=pl.BlockSpec((BLK, H), lambda i: (i, 0)),
        out_shape=jax.ShapeDtypeStruct((vp_rows, H), jnp.float32),
    )
    return f(tab, w.reshape(1, D), b.reshape(1, D), wt)


def _sc_pool_side_body(seg_total, n_per_w, g_hbm, id_hbm, o_hbm,
                       ids_v, rows_v, acc_v, sem):
    wid = lax.axis_index("s") * NC + lax.axis_index("c")
    base = wid * n_per_w
    for l in range(L):
        pltpu.sync_copy(id_hbm.at[pl.ds(l * seg_total + base, n_per_w)],
                        ids_v.at[l, pl.ds(0, n_per_w)])

    def chunk(ci, _):
        off = ci * CS
        descs = [
            pltpu.make_async_copy(
                g_hbm.at[ids_v.at[l, pl.ds(off, CS)]], rows_v.at[l], sem)
            for l in range(L)
        ]
        for dsc in descs:
            dsc.start()
        for dsc in descs:
            dsc.wait()

        def seg(si, _):
            for dc in range(H // 16):
                sl = pl.ds(dc * 16, 16)
                acc = rows_v[0, si, sl] + rows_v[1, si, sl]
                acc = acc + rows_v[2, si, sl]
                acc = acc + rows_v[3, si, sl]
                acc = acc + rows_v[4, si, sl]
                acc_v[si, sl] = acc
            return 0

        lax.fori_loop(0, CS, seg, 0)
        pltpu.sync_copy(acc_v, o_hbm.at[pl.ds(base + off, CS)])
        return 0

    lax.fori_loop(0, n_per_w // CS, chunk, 0)


@functools.cache
def _sc_pool_side(seg_total, n_per_w):
    return pl.kernel(
        functools.partial(_sc_pool_side_body, seg_total, n_per_w),
        mesh=plsc.VectorSubcoreMesh(core_axis_name="c", subcore_axis_name="s"),
        out_type=jax.ShapeDtypeStruct((seg_total, H), jnp.float32),
        scratch_types=[
            pltpu.VMEM((L, n_per_w), jnp.int32),
            pltpu.VMEM((L, CS, H), jnp.float32),
            pltpu.VMEM((CS, H), jnp.float32),
            pltpu.SemaphoreType.DMA,
        ],
        compiler_params=pltpu.CompilerParams(use_tc_tiling_on_sc=False),
    )


def _assemble_body(xnum_ref, vsum_ref, hsum_ref, xm_ref, nm_ref, cm_ref,
                   ncp_ref, colp_ref, nbias_ref, wt_ref, emb_ref, bert_ref):
    eps = 1e-12
    nden = jnp.sum(nm_ref[...].astype(jnp.float32), axis=1, keepdims=True) + eps
    ncp_avg = ncp_ref[...] / nden
    cden = jnp.sum(cm_ref[...].astype(jnp.float32), axis=1, keepdims=True) + eps
    colp_avg = colp_ref[...] / cden
    bias_p = jnp.dot(nbias_ref[...], wt_ref[...],
                     preferred_element_type=jnp.float32)
    rden = 1.0 / (jnp.sum(xm_ref[...].astype(jnp.float32), axis=2) + eps)
    val_avg = vsum_ref[...] * rden[:, :, None]
    hdr_avg = hsum_ref[...] * rden[:, :, None]
    num_part = xnum_ref[...][:, :, None] * ncp_avg[None] + bias_p[None]
    cat_part = (colp_avg[None] + val_avg) * 0.5
    emb_ref[...] = jnp.concatenate([num_part, cat_part], axis=1)
    bert_ref[...] = hdr_avg


def _assemble(xnum, vsum, hsum, xm, nm, cm, ncp, colp, nbias, wt):
    BB = 128
    nb = B // BB
    f = pl.pallas_call(
        _assemble_body,
        grid=(nb,),
        in_specs=[
            pl.BlockSpec((BB, N_NUM), lambda i: (i, 0)),
            pl.BlockSpec((BB, N_CAT, H), lambda i: (i, 0, 0)),
            pl.BlockSpec((BB, N_CAT, H), lambda i: (i, 0, 0)),
            pl.BlockSpec((BB, N_CAT, L), lambda i: (i, 0, 0)),
            pl.BlockSpec((N_NUM, L), lambda i: (0, 0)),
            pl.BlockSpec((N_CAT, L), lambda i: (0, 0)),
            pl.BlockSpec((N_NUM, H), lambda i: (0, 0)),
            pl.BlockSpec((N_CAT, H), lambda i: (0, 0)),
            pl.BlockSpec((1, D), lambda i: (0, 0)),
            pl.BlockSpec((D, H), lambda i: (0, 0)),
        ],
        out_specs=[
            pl.BlockSpec((BB, N_NUM + N_CAT, H), lambda i: (i, 0, 0)),
            pl.BlockSpec((BB, N_CAT, H), lambda i: (i, 0, 0)),
        ],
        out_shape=[
            jax.ShapeDtypeStruct((B, N_NUM + N_CAT, H), jnp.float32),
            jax.ShapeDtypeStruct((B, N_CAT, H), jnp.float32),
        ],
    )
    return f(xnum, vsum, hsum, xm, nm, cm, ncp, colp, nbias, wt)


def kernel(x_num, num_col_input_ids, num_att_mask, x_cat_input_ids,
           x_cat_att_mask, col_cat_input_ids, col_cat_att_mask, header_table,
           value_table, norm_header_w, norm_header_b, norm_value_w,
           norm_value_b, num_bias, align_W):
    wt = align_W.T
    gv = _table_proj(value_table, norm_value_w, norm_value_b, wt, VV, VPV)
    gh = _table_proj(header_table, norm_header_w, norm_header_b, wt, VH, VPH)

    xm = x_cat_att_mask != 0
    spread = (jnp.arange(SV * L, dtype=jnp.int32) & 511).reshape(B, N_CAT, L)
    val_ids = jnp.where(xm, x_cat_input_ids, VV + spread).reshape(SV, L)
    hdr_ids = jnp.where(xm, x_cat_input_ids, VH + spread).reshape(SV, L)
    spread_n = (jnp.arange(N_NUM * L, dtype=jnp.int32) & 511).reshape(N_NUM, L)
    spread_c = (jnp.arange(N_CAT * L, dtype=jnp.int32) & 511).reshape(N_CAT, L)
    nids = jnp.where(num_att_mask != 0, num_col_input_ids, VH + spread_n)
    cids = jnp.where(col_cat_att_mask != 0, col_cat_input_ids, VH + spread_c)
    npad = SH - SV - N_NUM - N_CAT
    pad = VH + (jnp.arange(npad * L, dtype=jnp.int32) & 511).reshape(npad, L)
    hdr_all = jnp.concatenate([hdr_ids, nids, cids, pad], axis=0)
    idv = val_ids.T.reshape(-1)
    idh = hdr_all.T.reshape(-1)

    vsum = _sc_pool_side(SV, NV)(gv, idv)
    hsum_all = _sc_pool_side(SH, NH)(gh, idh)
    vsum = vsum.reshape(B, N_CAT, H)
    hsum = hsum_all[:SV].reshape(B, N_CAT, H)
    ncp_sum = hsum_all[SV:SV + N_NUM]
    colp_sum = hsum_all[SV + N_NUM:SV + N_NUM + N_CAT]

    emb, bert = _assemble(x_num, vsum, hsum, x_cat_att_mask, num_att_mask,
                          col_cat_att_mask, ncp_sum, colp_sum,
                          num_bias.reshape(1, D), wt)
    return emb, bert
